# baseline (speedup 1.0000x reference)
import functools
import math

import jax
import jax.numpy as jnp
from jax import lax
from jax.experimental import pallas as pl
from jax.experimental.pallas import tpu as pltpu
from jax.experimental.pallas import tpu_sc as plsc

N = 10000
EDGES = 320000
D = 128
NMP = 2
NSC = 2
NTILES = 16
K = 96
CHUNKS = 216
QS = 24
NGR = QS // 3
EPT = CHUNKS * K
EPC = NTILES * EPT
DUMP = N
NROWS = 10008
WSTRIPE = 624
GK = 128
GCH = 5
GPT = GCH * GK
BM = 2000

_f32 = jnp.float32
_MESH = dict(core_axis_name="c", subcore_axis_name="s",
             num_cores=NSC, num_subcores=NTILES)


def _gather_count_body(e_hbm, eids_hbm, dst_hbm, h0_hbm, cnt_hbm,
                       cnt_sh, idx_v, rows_v, dst_v, ones_v, sem):
    cid = lax.axis_index("c")
    sid = lax.axis_index("s")

    @pl.loop(0, GK * (D // 16))
    def _fill(i):
        r = i // (D // 16)
        col = pl.ds((i % (D // 16)) * 16, 16)
        rows_v[r, col] = jnp.zeros((16,), _f32)

    @pl.loop(0, K * (D // 16))
    def _fill2(i):
        ones_v[i // (D // 16), pl.ds((i % (D // 16)) * 16, 16)] = (
            jnp.ones((16,), _f32))

    zbase = sid * WSTRIPE

    @pl.loop(0, WSTRIPE // GK)
    def _zstripe(k):
        pltpu.sync_copy(rows_v, cnt_sh.at[pl.ds(zbase + k * GK, GK)])

    zdone = (WSTRIPE // GK) * GK

    @pl.when(sid < NTILES - 1)
    def _zrem():
        pltpu.sync_copy(rows_v.at[pl.ds(0, WSTRIPE - zdone)],
                        cnt_sh.at[pl.ds(zbase + zdone, WSTRIPE - zdone)])

    @pl.when(sid == NTILES - 1)
    def _zrem_last():
        pltpu.sync_copy(rows_v.at[pl.ds(0, NROWS - 15 * WSTRIPE - zdone)],
                        cnt_sh.at[pl.ds(zbase + zdone,
                                        NROWS - 15 * WSTRIPE - zdone)])

    pltpu.sync_copy(eids_hbm.at[cid, sid], idx_v)
    base = sid * WSTRIPE
    for j in range(GCH - 1):
        pltpu.async_copy(e_hbm.at[idx_v.at[j]], rows_v, sem).wait()
        pltpu.sync_copy(rows_v, h0_hbm.at[cid, pl.ds(base + j * GK, GK)])
    pltpu.async_copy(e_hbm.at[idx_v.at[GCH - 1]], rows_v, sem).wait()
    tail = WSTRIPE - (GCH - 1) * GK

    @pl.when(sid < NTILES - 1)
    def _w_tail():
        pltpu.sync_copy(rows_v.at[pl.ds(0, tail)],
                        h0_hbm.at[cid, pl.ds(base + (GCH - 1) * GK, tail)])

    @pl.when(sid == NTILES - 1)
    def _w_tail_last():
        pltpu.sync_copy(rows_v,
                        h0_hbm.at[cid, pl.ds(base + (GCH - 1) * GK, GK)])

    plsc.subcore_barrier()

    for h in range(CHUNKS // QS):
        pltpu.sync_copy(dst_hbm.at[cid, sid, pl.ds(h * QS, QS)], dst_v)

        @pl.loop(0, QS)
        def _count(j):
            pltpu.sync_copy(ones_v, cnt_sh.at[dst_v.at[j]], add=True)

    plsc.subcore_barrier()
    pltpu.sync_copy(cnt_sh.at[pl.ds(base, WSTRIPE)],
                    cnt_hbm.at[cid, pl.ds(base, WSTRIPE)])

    @pl.when(sid == NTILES - 1)
    def _w_cnt_last():
        pltpu.sync_copy(cnt_sh.at[pl.ds(NTILES * WSTRIPE, N - NTILES * WSTRIPE)],
                        cnt_hbm.at[cid, pl.ds(NTILES * WSTRIPE,
                                              N - NTILES * WSTRIPE)])


_sc_gather_count = functools.partial(
    pl.kernel,
    out_type=(jax.ShapeDtypeStruct((NMP, N, D), _f32),
              jax.ShapeDtypeStruct((NMP, N, D), _f32)),
    mesh=plsc.VectorSubcoreMesh(**_MESH),
    scratch_types=[
        pltpu.VMEM_SHARED((NROWS, D), _f32),
        pltpu.VMEM((GCH, GK), jnp.int32),
        pltpu.VMEM((GK, D), _f32),
        pltpu.VMEM((QS, K), jnp.int32),
        pltpu.VMEM((K, D), _f32),
        pltpu.SemaphoreType.DMA,
    ],
)(_gather_count_body)


def _edge_agg_body(y_hbm, src_hbm, dst_hbm, agg_hbm,
                   agg_sh, src_v, dst_v, b0, b1, b2, s0, s1, s2):
    cid = lax.axis_index("c")
    sid = lax.axis_index("s")

    @pl.loop(0, K * (D // 16))
    def _zfill(i):
        b0[i // (D // 16), pl.ds((i % (D // 16)) * 16, 16)] = (
            jnp.zeros((16,), _f32))

    zbase = sid * WSTRIPE

    @pl.loop(0, WSTRIPE // K)
    def _zstripe(k):
        pltpu.sync_copy(b0, agg_sh.at[pl.ds(zbase + k * K, K)])

    zdone = (WSTRIPE // K) * K

    @pl.when(sid < NTILES - 1)
    def _zrem():
        pltpu.sync_copy(b0.at[pl.ds(0, WSTRIPE - zdone)],
                        agg_sh.at[pl.ds(zbase + zdone, WSTRIPE - zdone)])

    @pl.when(sid == NTILES - 1)
    def _zrem_last():
        pltpu.sync_copy(b0.at[pl.ds(0, NROWS - 15 * WSTRIPE - zdone)],
                        agg_sh.at[pl.ds(zbase + zdone,
                                        NROWS - 15 * WSTRIPE - zdone)])

    plsc.subcore_barrier()

    def _fire(j, buf, sem):
        pltpu.async_copy(y_hbm.at[src_v.at[j]], buf, sem)

    def _wait(buf, sem):
        pltpu.make_async_copy(y_hbm.at[pl.ds(0, K)], buf, sem).wait()

    def _scat(j, buf):
        pltpu.sync_copy(buf, agg_sh.at[dst_v.at[j]], add=True)

    for q in range(CHUNKS // QS):
        pltpu.sync_copy(src_hbm.at[cid, sid, pl.ds(q * QS, QS)], src_v)
        pltpu.sync_copy(dst_hbm.at[cid, sid, pl.ds(q * QS, QS)], dst_v)
        _fire(0, b0, s0)
        _fire(1, b1, s1)

        @pl.loop(0, NGR)
        def _ring(g):
            j = 3 * g
            _fire(jnp.minimum(j + 2, QS - 1), b2, s2)
            _wait(b0, s0)
            _scat(j, b0)
            _fire(jnp.minimum(j + 3, QS - 1), b0, s0)
            _wait(b1, s1)
            _scat(j + 1, b1)
            _fire(jnp.minimum(j + 4, QS - 1), b1, s1)
            _wait(b2, s2)
            _scat(j + 2, b2)

        _wait(b0, s0)
        _wait(b1, s1)

    plsc.subcore_barrier()
    pltpu.sync_copy(agg_sh.at[pl.ds(sid * WSTRIPE, WSTRIPE)],
                    agg_hbm.at[cid, pl.ds(sid * WSTRIPE, WSTRIPE)])

    @pl.when(sid == NTILES - 1)
    def _w_last():
        pltpu.sync_copy(agg_sh.at[pl.ds(NTILES * WSTRIPE, N - NTILES * WSTRIPE)],
                        agg_hbm.at[cid, pl.ds(NTILES * WSTRIPE,
                                              N - NTILES * WSTRIPE)])


_sc_edge_agg = functools.partial(
    pl.kernel,
    out_type=jax.ShapeDtypeStruct((NMP, N, D), _f32),
    mesh=plsc.VectorSubcoreMesh(**_MESH),
    scratch_types=[
        pltpu.VMEM_SHARED((NROWS, D), _f32),
        pltpu.VMEM((QS, K), jnp.int32),
        pltpu.VMEM((QS, K), jnp.int32),
        pltpu.VMEM((K, D), _f32),
        pltpu.VMEM((K, D), _f32),
        pltpu.VMEM((K, D), _f32),
        pltpu.SemaphoreType.DMA,
        pltpu.SemaphoreType.DMA,
        pltpu.SemaphoreType.DMA,
    ],
)(_edge_agg_body)


def _mm_body(h_ref, wr_ref, wt_ref, b_ref, y_ref, z_ref):
    h = h_ref[0]
    b = jnp.where(pl.program_id(0) == 0, b_ref[0:1, :], b_ref[1:2, :])
    y_ref[...] = jnp.dot(h, wr_ref[0], preferred_element_type=_f32)
    z_ref[0] = jnp.dot(h, wt_ref[0], preferred_element_type=_f32) + b


_tc_mm = pl.pallas_call(
    _mm_body,
    grid=(NMP, N // BM),
    in_specs=[
        pl.BlockSpec((1, BM, D), lambda c, m: (c, m, 0)),
        pl.BlockSpec((1, D, D), lambda c, m: (c, 0, 0)),
        pl.BlockSpec((1, D, D), lambda c, m: (c, 0, 0)),
        pl.BlockSpec((NMP, D), lambda c, m: (0, 0)),
    ],
    out_specs=[
        pl.BlockSpec((BM, D), lambda c, m: (c * (N // BM) + m, 0)),
        pl.BlockSpec((1, BM, D), lambda c, m: (c, m, 0)),
    ],
    out_shape=[
        jax.ShapeDtypeStruct((NMP * N, D), _f32),
        jax.ShapeDtypeStruct((NMP, N, D), _f32),
    ],
)


def _comb_mm_body(agg_ref, cnt_ref, z0_ref, wr_ref, wt_ref, b_ref,
                  y_ref, z_ref):
    inv = 1.0 / jnp.maximum(cnt_ref[0][:, 0:1], 1.0)
    h = jnp.maximum(agg_ref[0] * inv + z0_ref[0], 0.0)
    b = jnp.where(pl.program_id(0) == 0, b_ref[0:1, :], b_ref[1:2, :])
    y_ref[...] = jnp.dot(h, wr_ref[0], preferred_element_type=_f32)
    z_ref[0] = jnp.dot(h, wt_ref[0], preferred_element_type=_f32) + b


_tc_comb_mm = pl.pallas_call(
    _comb_mm_body,
    grid=(NMP, N // BM),
    in_specs=[
        pl.BlockSpec((1, BM, D), lambda c, m: (c, m, 0)),
        pl.BlockSpec((1, BM, D), lambda c, m: (c, m, 0)),
        pl.BlockSpec((1, BM, D), lambda c, m: (c, m, 0)),
        pl.BlockSpec((1, D, D), lambda c, m: (c, 0, 0)),
        pl.BlockSpec((1, D, D), lambda c, m: (c, 0, 0)),
        pl.BlockSpec((NMP, D), lambda c, m: (0, 0)),
    ],
    out_specs=[
        pl.BlockSpec((BM, D), lambda c, m: (c * (N // BM) + m, 0)),
        pl.BlockSpec((1, BM, D), lambda c, m: (c, m, 0)),
    ],
    out_shape=[
        jax.ShapeDtypeStruct((NMP * N, D), _f32),
        jax.ShapeDtypeStruct((NMP, N, D), _f32),
    ],
)


def _fuse_body(agg_ref, cnt_ref, z1_ref, meta_ref, wqt_ref, bq_ref, o_ref):
    q = jnp.dot(meta_ref[...], wqt_ref[...],
                preferred_element_type=_f32) + bq_ref[...]
    inv0 = 1.0 / jnp.maximum(cnt_ref[0][:, 0:1], 1.0)
    inv1 = 1.0 / jnp.maximum(cnt_ref[1][:, 0:1], 1.0)
    h0 = jnp.maximum(agg_ref[0] * inv0 + z1_ref[0], 0.0)
    h1 = jnp.maximum(agg_ref[1] * inv1 + z1_ref[1], 0.0)
    scale = 1.0 / math.sqrt(D)
    s0 = jnp.sum(h0 * q[0:1, :], axis=1, keepdims=True) * scale
    s1 = jnp.sum(h1 * q[1:2, :], axis=1, keepdims=True) * scale
    w0 = 1.0 / (1.0 + jnp.exp(s1 - s0))
    o_ref[...] = w0 * h0 + (1.0 - w0) * h1


_tc_fuse = pl.pallas_call(
    _fuse_body,
    grid=(N // BM,),
    in_specs=[
        pl.BlockSpec((NMP, BM, D), lambda m: (0, m, 0)),
        pl.BlockSpec((NMP, BM, D), lambda m: (0, m, 0)),
        pl.BlockSpec((NMP, BM, D), lambda m: (0, m, 0)),
        pl.BlockSpec((NMP, 64), lambda m: (0, 0)),
        pl.BlockSpec((64, D), lambda m: (0, 0)),
        pl.BlockSpec((1, D), lambda m: (0, 0)),
    ],
    out_specs=pl.BlockSpec((BM, D), lambda m: (m, 0)),
    out_shape=jax.ShapeDtypeStruct((N, D), _f32),
)


def _prep_edges(ei, c):
    src = ei[0].astype(jnp.int32) + jnp.int32(c * N)
    dst = ei[1].astype(jnp.int32)
    pad = EPC - EDGES
    src = jnp.concatenate([src, jnp.zeros((pad,), jnp.int32)])
    dst = jnp.concatenate([dst, jnp.full((pad,), DUMP, jnp.int32)])
    return src.reshape(NTILES, CHUNKS, K), dst.reshape(NTILES, CHUNKS, K)


def kernel(E, edge_index0, eids0, edge_index1, eids1, metapath_emb,
           ifdropout, W_rel_0, W_root_0, b_0, W_rel_1, W_root_1, b_1,
           W_rel_2, W_root_2, b_2, W_rel_3, W_root_3, b_3, Wq, bq):
    eids_all = jnp.stack([eids0, eids1]).astype(jnp.int32)
    eids = jnp.stack([eids_all[:, s * WSTRIPE:s * WSTRIPE + GPT]
                      for s in range(NTILES)], axis=1)
    eids = eids.reshape(NMP, NTILES, GCH, GK)

    s0, d0 = _prep_edges(edge_index0, 0)
    s1, d1 = _prep_edges(edge_index1, 1)
    src_r = jnp.stack([s0, s1])
    dst_r = jnp.stack([d0, d1])

    Wr0 = jnp.stack([W_rel_0[0], W_rel_2[0]])
    Wt0 = jnp.stack([W_root_0, W_root_2])
    bb0 = jnp.stack([b_0, b_2])
    Wr1 = jnp.stack([W_rel_1[0], W_rel_3[0]])
    Wt1 = jnp.stack([W_root_1, W_root_3])
    bb1 = jnp.stack([b_1, b_3])
    WqT = Wq.T
    bq2 = bq.reshape(1, D)

    h0, cnt = _sc_gather_count(E, eids, dst_r)
    y0, z0 = _tc_mm(h0, Wr0, Wt0, bb0)
    agg0 = _sc_edge_agg(y0, src_r, dst_r)
    y1, z1 = _tc_comb_mm(agg0, cnt, z0, Wr1, Wt1, bb1)
    agg1 = _sc_edge_agg(y1, src_r, dst_r)
    return _tc_fuse(agg1, cnt, z1, metapath_emb, WqT, bq2)


# speedup vs baseline: 1.3753x; 1.3753x over previous
import functools
import math

import jax
import jax.numpy as jnp
from jax import lax
from jax.experimental import pallas as pl
from jax.experimental.pallas import tpu as pltpu
from jax.experimental.pallas import tpu_sc as plsc

N = 10000
EDGES = 320000
D = 128
NMP = 2
NSC = 2
NTILES = 16
K = 128
CHUNKS = 160
HALF = CHUNKS // 2
QC = 40
SUP = 80
SSUP = 16
EPT = CHUNKS * K
EPC = NTILES * EPT
DUMP = N
ZSTRIPE = 632
NROWS = NTILES * ZSTRIPE
WSTRIPE = 624
GCH = 5
GPT = GCH * K
BM = 2000

_f32 = jnp.float32
_MESH = dict(core_axis_name="c", subcore_axis_name="s",
             num_cores=NSC, num_subcores=NTILES)


def _gather_count_body(e_hbm, eids_hbm, dst_hbm, h0_hbm, cnt_hbm,
                       cnt_sh, idx_v, rows_v, dst_v, ones_v, sem):
    cid = lax.axis_index("c")
    sid = lax.axis_index("s")

    @pl.loop(0, K * (D // 16))
    def _fill(i):
        r = i // (D // 16)
        col = pl.ds((i % (D // 16)) * 16, 16)
        rows_v[r, col] = jnp.zeros((16,), _f32)
        ones_v[r, col] = jnp.ones((16,), _f32)

    zbase = sid * ZSTRIPE

    @pl.loop(0, ZSTRIPE // K)
    def _zstripe(k):
        pltpu.sync_copy(rows_v, cnt_sh.at[pl.ds(zbase + k * K, K)])

    rem = ZSTRIPE - (ZSTRIPE // K) * K
    pltpu.sync_copy(rows_v.at[pl.ds(0, rem)],
                    cnt_sh.at[pl.ds(zbase + (ZSTRIPE // K) * K, rem)])

    pltpu.sync_copy(eids_hbm.at[cid, sid], idx_v)
    base = sid * WSTRIPE
    for j in range(GCH - 1):
        pltpu.async_copy(e_hbm.at[idx_v.at[j]], rows_v, sem).wait()
        pltpu.sync_copy(rows_v, h0_hbm.at[cid, pl.ds(base + j * K, K)])
    pltpu.async_copy(e_hbm.at[idx_v.at[GCH - 1]], rows_v, sem).wait()
    tail = WSTRIPE - (GCH - 1) * K

    @pl.when(sid < NTILES - 1)
    def _w_tail():
        pltpu.sync_copy(rows_v.at[pl.ds(0, tail)],
                        h0_hbm.at[cid, pl.ds(base + (GCH - 1) * K, tail)])

    @pl.when(sid == NTILES - 1)
    def _w_tail_last():
        pltpu.sync_copy(rows_v,
                        h0_hbm.at[cid, pl.ds(base + (GCH - 1) * K, K)])

    plsc.subcore_barrier()

    for h in range(2):
        pltpu.sync_copy(dst_hbm.at[cid, sid, pl.ds(h * HALF, HALF)], dst_v)

        @pl.loop(0, HALF)
        def _count(j):
            pltpu.sync_copy(ones_v, cnt_sh.at[dst_v.at[j]], add=True)

    plsc.subcore_barrier()
    pltpu.sync_copy(cnt_sh.at[pl.ds(base, WSTRIPE)],
                    cnt_hbm.at[cid, pl.ds(base, WSTRIPE)])

    @pl.when(sid == NTILES - 1)
    def _w_cnt_last():
        pltpu.sync_copy(cnt_sh.at[pl.ds(NTILES * WSTRIPE, N - NTILES * WSTRIPE)],
                        cnt_hbm.at[cid, pl.ds(NTILES * WSTRIPE,
                                              N - NTILES * WSTRIPE)])


_sc_gather_count = functools.partial(
    pl.kernel,
    out_type=(jax.ShapeDtypeStruct((NMP, N, D), _f32),
              jax.ShapeDtypeStruct((NMP, N, D), _f32)),
    mesh=plsc.VectorSubcoreMesh(**_MESH),
    scratch_types=[
        pltpu.VMEM_SHARED((NROWS, D), _f32),
        pltpu.VMEM((GCH, K), jnp.int32),
        pltpu.VMEM((K, D), _f32),
        pltpu.VMEM((HALF, K), jnp.int32),
        pltpu.VMEM((K, D), _f32),
        pltpu.SemaphoreType.DMA,
    ],
)(_gather_count_body)


def _edge_agg_body(y_hbm, src_hbm, dst_hbm, agg_hbm,
                   agg_sh, src_v, dst_v, buf, sem):
    cid = lax.axis_index("c")
    sid = lax.axis_index("s")

    @pl.loop(0, K * (D // 16))
    def _zfill(i):
        buf[i // (D // 16), pl.ds((i % (D // 16)) * 16, 16)] = (
            jnp.zeros((16,), _f32))

    zbase = sid * ZSTRIPE

    @pl.loop(0, ZSTRIPE // K)
    def _zstripe(k):
        pltpu.sync_copy(buf.at[pl.ds(0, K)], agg_sh.at[pl.ds(zbase + k * K, K)])

    rem = ZSTRIPE - (ZSTRIPE // K) * K
    pltpu.sync_copy(buf.at[pl.ds(0, rem)],
                    agg_sh.at[pl.ds(zbase + (ZSTRIPE // K) * K, rem)])

    plsc.subcore_barrier()

    for st in range(SUP // SSUP):
        pltpu.sync_copy(src_hbm.at[cid, sid, pl.ds(st * SSUP, SSUP)], src_v)
        pltpu.sync_copy(dst_hbm.at[cid, sid, pl.ds(st * 2 * SSUP, 2 * SSUP)],
                        dst_v)

        @pl.loop(0, SSUP)
        def _supers(s):
            pltpu.async_copy(y_hbm.at[src_v.at[s]], buf, sem).wait()
            pltpu.sync_copy(buf.at[pl.ds(0, K)],
                            agg_sh.at[dst_v.at[2 * s]], add=True)
            pltpu.sync_copy(buf.at[pl.ds(K, K)],
                            agg_sh.at[dst_v.at[2 * s + 1]], add=True)

    plsc.subcore_barrier()
    pltpu.sync_copy(agg_sh.at[pl.ds(sid * WSTRIPE, WSTRIPE)],
                    agg_hbm.at[cid, pl.ds(sid * WSTRIPE, WSTRIPE)])

    @pl.when(sid == NTILES - 1)
    def _w_last():
        pltpu.sync_copy(agg_sh.at[pl.ds(NTILES * WSTRIPE, N - NTILES * WSTRIPE)],
                        agg_hbm.at[cid, pl.ds(NTILES * WSTRIPE,
                                              N - NTILES * WSTRIPE)])


_sc_edge_agg = functools.partial(
    pl.kernel,
    out_type=jax.ShapeDtypeStruct((NMP, N, D), _f32),
    mesh=plsc.VectorSubcoreMesh(**_MESH),
    scratch_types=[
        pltpu.VMEM_SHARED((NROWS, D), _f32),
        pltpu.VMEM((SSUP, 2 * K), jnp.int32),
        pltpu.VMEM((2 * SSUP, K), jnp.int32),
        pltpu.VMEM((2 * K, D), _f32),
        pltpu.SemaphoreType.DMA,
    ],
    compiler_params=pltpu.CompilerParams(use_tc_tiling_on_sc=False),
)(_edge_agg_body)


def _mm_body(h_ref, wr_ref, wt_ref, b_ref, y_ref, z_ref):
    h = h_ref[0]
    b = jnp.where(pl.program_id(0) == 0, b_ref[0:1, :], b_ref[1:2, :])
    y_ref[...] = jnp.dot(h, wr_ref[0], preferred_element_type=_f32)
    z_ref[0] = jnp.dot(h, wt_ref[0], preferred_element_type=_f32) + b


_tc_mm = pl.pallas_call(
    _mm_body,
    grid=(NMP, N // BM),
    in_specs=[
        pl.BlockSpec((1, BM, D), lambda c, m: (c, m, 0)),
        pl.BlockSpec((1, D, D), lambda c, m: (c, 0, 0)),
        pl.BlockSpec((1, D, D), lambda c, m: (c, 0, 0)),
        pl.BlockSpec((NMP, D), lambda c, m: (0, 0)),
    ],
    out_specs=[
        pl.BlockSpec((BM, D), lambda c, m: (c * (N // BM) + m, 0)),
        pl.BlockSpec((1, BM, D), lambda c, m: (c, m, 0)),
    ],
    out_shape=[
        jax.ShapeDtypeStruct((NMP * N, D), _f32),
        jax.ShapeDtypeStruct((NMP, N, D), _f32),
    ],
)


def _comb_mm_body(agg_ref, cnt_ref, z0_ref, wr_ref, wt_ref, b_ref,
                  y_ref, z_ref):
    inv = 1.0 / jnp.maximum(cnt_ref[0][:, 0:1], 1.0)
    h = jnp.maximum(agg_ref[0] * inv + z0_ref[0], 0.0)
    b = jnp.where(pl.program_id(0) == 0, b_ref[0:1, :], b_ref[1:2, :])
    y_ref[...] = jnp.dot(h, wr_ref[0], preferred_element_type=_f32)
    z_ref[0] = jnp.dot(h, wt_ref[0], preferred_element_type=_f32) + b


_tc_comb_mm = pl.pallas_call(
    _comb_mm_body,
    grid=(NMP, N // BM),
    in_specs=[
        pl.BlockSpec((1, BM, D), lambda c, m: (c, m, 0)),
        pl.BlockSpec((1, BM, D), lambda c, m: (c, m, 0)),
        pl.BlockSpec((1, BM, D), lambda c, m: (c, m, 0)),
        pl.BlockSpec((1, D, D), lambda c, m: (c, 0, 0)),
        pl.BlockSpec((1, D, D), lambda c, m: (c, 0, 0)),
        pl.BlockSpec((NMP, D), lambda c, m: (0, 0)),
    ],
    out_specs=[
        pl.BlockSpec((BM, D), lambda c, m: (c * (N // BM) + m, 0)),
        pl.BlockSpec((1, BM, D), lambda c, m: (c, m, 0)),
    ],
    out_shape=[
        jax.ShapeDtypeStruct((NMP * N, D), _f32),
        jax.ShapeDtypeStruct((NMP, N, D), _f32),
    ],
)


def _fuse_body(agg_ref, cnt_ref, z1_ref, meta_ref, wqt_ref, bq_ref, o_ref):
    q = jnp.dot(meta_ref[...], wqt_ref[...],
                preferred_element_type=_f32) + bq_ref[...]
    inv0 = 1.0 / jnp.maximum(cnt_ref[0][:, 0:1], 1.0)
    inv1 = 1.0 / jnp.maximum(cnt_ref[1][:, 0:1], 1.0)
    h0 = jnp.maximum(agg_ref[0] * inv0 + z1_ref[0], 0.0)
    h1 = jnp.maximum(agg_ref[1] * inv1 + z1_ref[1], 0.0)
    scale = 1.0 / math.sqrt(D)
    s0 = jnp.sum(h0 * q[0:1, :], axis=1, keepdims=True) * scale
    s1 = jnp.sum(h1 * q[1:2, :], axis=1, keepdims=True) * scale
    w0 = 1.0 / (1.0 + jnp.exp(s1 - s0))
    o_ref[...] = w0 * h0 + (1.0 - w0) * h1


_tc_fuse = pl.pallas_call(
    _fuse_body,
    grid=(N // BM,),
    in_specs=[
        pl.BlockSpec((NMP, BM, D), lambda m: (0, m, 0)),
        pl.BlockSpec((NMP, BM, D), lambda m: (0, m, 0)),
        pl.BlockSpec((NMP, BM, D), lambda m: (0, m, 0)),
        pl.BlockSpec((NMP, 64), lambda m: (0, 0)),
        pl.BlockSpec((64, D), lambda m: (0, 0)),
        pl.BlockSpec((1, D), lambda m: (0, 0)),
    ],
    out_specs=pl.BlockSpec((BM, D), lambda m: (m, 0)),
    out_shape=jax.ShapeDtypeStruct((N, D), _f32),
)


def _prep_edges(ei, c):
    src = ei[0].astype(jnp.int32) + jnp.int32(c * N)
    dst = ei[1].astype(jnp.int32)
    pad = EPC - EDGES
    src = jnp.concatenate([src, jnp.zeros((pad,), jnp.int32)])
    dst = jnp.concatenate([dst, jnp.full((pad,), DUMP, jnp.int32)])
    return (src.reshape(NTILES, SUP, 2 * K),
            dst.reshape(NTILES, CHUNKS, K))


def kernel(E, edge_index0, eids0, edge_index1, eids1, metapath_emb,
           ifdropout, W_rel_0, W_root_0, b_0, W_rel_1, W_root_1, b_1,
           W_rel_2, W_root_2, b_2, W_rel_3, W_root_3, b_3, Wq, bq):
    eids_all = jnp.stack([eids0, eids1]).astype(jnp.int32)
    eids = jnp.stack([eids_all[:, s * WSTRIPE:s * WSTRIPE + GPT]
                      for s in range(NTILES)], axis=1)
    eids = eids.reshape(NMP, NTILES, GCH, K)

    s0, d0 = _prep_edges(edge_index0, 0)
    s1, d1 = _prep_edges(edge_index1, 1)
    src_r = jnp.stack([s0, s1])
    dst_r = jnp.stack([d0, d1])

    Wr0 = jnp.stack([W_rel_0[0], W_rel_2[0]])
    Wt0 = jnp.stack([W_root_0, W_root_2])
    bb0 = jnp.stack([b_0, b_2])
    Wr1 = jnp.stack([W_rel_1[0], W_rel_3[0]])
    Wt1 = jnp.stack([W_root_1, W_root_3])
    bb1 = jnp.stack([b_1, b_3])
    WqT = Wq.T
    bq2 = bq.reshape(1, D)

    h0, cnt = _sc_gather_count(E, eids, dst_r)
    y0, z0 = _tc_mm(h0, Wr0, Wt0, bb0)
    agg0 = _sc_edge_agg(y0, src_r, dst_r)
    y1, z1 = _tc_comb_mm(agg0, cnt, z0, Wr1, Wt1, bb1)
    agg1 = _sc_edge_agg(y1, src_r, dst_r)
    return _tc_fuse(agg1, cnt, z1, metapath_emb, WqT, bq2)


# speedup vs baseline: 1.4257x; 1.0367x over previous
import functools
import math

import jax
import jax.numpy as jnp
from jax import lax
from jax.experimental import pallas as pl
from jax.experimental.pallas import tpu as pltpu
from jax.experimental.pallas import tpu_sc as plsc

N = 10000
EDGES = 320000
D = 128
NMP = 2
NSC = 2
NTILES = 16
K = 128
CHUNKS = 160
HALF = CHUNKS // 2
QC = 40
EPT = CHUNKS * K
EPC = NTILES * EPT
DUMP = N
ZSTRIPE = 632
NROWS = NTILES * ZSTRIPE
WSTRIPE = 624
GCH = 5
GPT = GCH * K
BM = 2000

_f32 = jnp.float32
_MESH = dict(core_axis_name="c", subcore_axis_name="s",
             num_cores=NSC, num_subcores=NTILES)


def _gather_count_body(e_hbm, eids_hbm, dst_hbm, h0_hbm, cnt_hbm,
                       cnt_sh, idx_v, rows_v, dst_v, ones_v, sem):
    cid = lax.axis_index("c")
    sid = lax.axis_index("s")

    @pl.loop(0, K * (D // 16))
    def _fill(i):
        r = i // (D // 16)
        col = pl.ds((i % (D // 16)) * 16, 16)
        rows_v[r, col] = jnp.zeros((16,), _f32)
        ones_v[r, col] = jnp.ones((16,), _f32)

    zbase = sid * ZSTRIPE

    @pl.loop(0, ZSTRIPE // K)
    def _zstripe(k):
        pltpu.sync_copy(rows_v, cnt_sh.at[pl.ds(zbase + k * K, K)])

    rem = ZSTRIPE - (ZSTRIPE // K) * K
    pltpu.sync_copy(rows_v.at[pl.ds(0, rem)],
                    cnt_sh.at[pl.ds(zbase + (ZSTRIPE // K) * K, rem)])

    pltpu.sync_copy(eids_hbm.at[cid, sid], idx_v)
    base = sid * WSTRIPE
    for j in range(GCH - 1):
        pltpu.async_copy(e_hbm.at[idx_v.at[j]], rows_v, sem).wait()
        pltpu.sync_copy(rows_v, h0_hbm.at[cid, pl.ds(base + j * K, K)])
    pltpu.async_copy(e_hbm.at[idx_v.at[GCH - 1]], rows_v, sem).wait()
    tail = WSTRIPE - (GCH - 1) * K

    @pl.when(sid < NTILES - 1)
    def _w_tail():
        pltpu.sync_copy(rows_v.at[pl.ds(0, tail)],
                        h0_hbm.at[cid, pl.ds(base + (GCH - 1) * K, tail)])

    @pl.when(sid == NTILES - 1)
    def _w_tail_last():
        pltpu.sync_copy(rows_v,
                        h0_hbm.at[cid, pl.ds(base + (GCH - 1) * K, K)])

    plsc.subcore_barrier()

    for h in range(2):
        pltpu.sync_copy(dst_hbm.at[cid, sid, pl.ds(h * HALF, HALF)], dst_v)

        @pl.loop(0, HALF // 8)
        def _count(g):
            for t in range(8):
                pltpu.async_copy(ones_v, cnt_sh.at[dst_v.at[8 * g + t]],
                                 sem, add=True)
            for t in range(8):
                pltpu.make_async_copy(ones_v, cnt_sh.at[dst_v.at[8 * g]],
                                      sem).wait()

    plsc.subcore_barrier()
    pltpu.sync_copy(cnt_sh.at[pl.ds(base, WSTRIPE)],
                    cnt_hbm.at[cid, pl.ds(base, WSTRIPE)])

    @pl.when(sid == NTILES - 1)
    def _w_cnt_last():
        pltpu.sync_copy(cnt_sh.at[pl.ds(NTILES * WSTRIPE, N - NTILES * WSTRIPE)],
                        cnt_hbm.at[cid, pl.ds(NTILES * WSTRIPE,
                                              N - NTILES * WSTRIPE)])


_sc_gather_count = functools.partial(
    pl.kernel,
    out_type=(jax.ShapeDtypeStruct((NMP, N, D), _f32),
              jax.ShapeDtypeStruct((NMP, N, D), _f32)),
    mesh=plsc.VectorSubcoreMesh(**_MESH),
    scratch_types=[
        pltpu.VMEM_SHARED((NROWS, D), _f32),
        pltpu.VMEM((GCH, K), jnp.int32),
        pltpu.VMEM((K, D), _f32),
        pltpu.VMEM((HALF, K), jnp.int32),
        pltpu.VMEM((K, D), _f32),
        pltpu.SemaphoreType.DMA,
    ],
)(_gather_count_body)


def _edge_agg_body(y_hbm, src_hbm, dst_hbm, agg_hbm,
                   agg_sh, src_v, dst_v, bufa, bufb, sema, semb):
    cid = lax.axis_index("c")
    sid = lax.axis_index("s")

    @pl.loop(0, K * (D // 16))
    def _zfill(i):
        bufa[i // (D // 16), pl.ds((i % (D // 16)) * 16, 16)] = (
            jnp.zeros((16,), _f32))

    zbase = sid * ZSTRIPE

    @pl.loop(0, ZSTRIPE // K)
    def _zstripe(k):
        pltpu.sync_copy(bufa, agg_sh.at[pl.ds(zbase + k * K, K)])

    rem = ZSTRIPE - (ZSTRIPE // K) * K
    pltpu.sync_copy(bufa.at[pl.ds(0, rem)],
                    agg_sh.at[pl.ds(zbase + (ZSTRIPE // K) * K, rem)])

    plsc.subcore_barrier()

    def _wait(buf, sem):
        pltpu.make_async_copy(y_hbm.at[pl.ds(0, K)], buf, sem).wait()

    for q in range(CHUNKS // QC):
        pltpu.sync_copy(src_hbm.at[cid, sid, pl.ds(q * QC, QC)], src_v)
        pltpu.sync_copy(dst_hbm.at[cid, sid, pl.ds(q * QC, QC)], dst_v)
        pltpu.async_copy(y_hbm.at[src_v.at[0]], bufa, sema)

        @pl.loop(0, QC // 2)
        def _pairs(p):
            j0 = 2 * p
            pltpu.async_copy(y_hbm.at[src_v.at[j0 + 1]], bufb, semb)
            _wait(bufa, sema)
            pltpu.sync_copy(bufa, agg_sh.at[dst_v.at[j0]], add=True)
            jn = jnp.minimum(j0 + 2, QC - 1)
            pltpu.async_copy(y_hbm.at[src_v.at[jn]], bufa, sema)
            _wait(bufb, semb)
            pltpu.sync_copy(bufb, agg_sh.at[dst_v.at[j0 + 1]], add=True)

        _wait(bufa, sema)

    plsc.subcore_barrier()
    pltpu.sync_copy(agg_sh.at[pl.ds(sid * WSTRIPE, WSTRIPE)],
                    agg_hbm.at[cid, pl.ds(sid * WSTRIPE, WSTRIPE)])

    @pl.when(sid == NTILES - 1)
    def _w_last():
        pltpu.sync_copy(agg_sh.at[pl.ds(NTILES * WSTRIPE, N - NTILES * WSTRIPE)],
                        agg_hbm.at[cid, pl.ds(NTILES * WSTRIPE,
                                              N - NTILES * WSTRIPE)])


_sc_edge_agg = functools.partial(
    pl.kernel,
    out_type=jax.ShapeDtypeStruct((NMP, N, D), _f32),
    mesh=plsc.VectorSubcoreMesh(**_MESH),
    scratch_types=[
        pltpu.VMEM_SHARED((NROWS, D), _f32),
        pltpu.VMEM((QC, K), jnp.int32),
        pltpu.VMEM((QC, K), jnp.int32),
        pltpu.VMEM((K, D), _f32),
        pltpu.VMEM((K, D), _f32),
        pltpu.SemaphoreType.DMA,
        pltpu.SemaphoreType.DMA,
    ],
)(_edge_agg_body)


def _mm_body(h_ref, wr_ref, wt_ref, b_ref, y_ref, z_ref):
    h = h_ref[0]
    b = jnp.where(pl.program_id(0) == 0, b_ref[0:1, :], b_ref[1:2, :])
    y_ref[...] = jnp.dot(h, wr_ref[0], preferred_element_type=_f32)
    z_ref[0] = jnp.dot(h, wt_ref[0], preferred_element_type=_f32) + b


_tc_mm = pl.pallas_call(
    _mm_body,
    grid=(NMP, N // BM),
    in_specs=[
        pl.BlockSpec((1, BM, D), lambda c, m: (c, m, 0)),
        pl.BlockSpec((1, D, D), lambda c, m: (c, 0, 0)),
        pl.BlockSpec((1, D, D), lambda c, m: (c, 0, 0)),
        pl.BlockSpec((NMP, D), lambda c, m: (0, 0)),
    ],
    out_specs=[
        pl.BlockSpec((BM, D), lambda c, m: (c * (N // BM) + m, 0)),
        pl.BlockSpec((1, BM, D), lambda c, m: (c, m, 0)),
    ],
    out_shape=[
        jax.ShapeDtypeStruct((NMP * N, D), _f32),
        jax.ShapeDtypeStruct((NMP, N, D), _f32),
    ],
)


def _comb_mm_body(agg_ref, cnt_ref, z0_ref, wr_ref, wt_ref, b_ref,
                  y_ref, z_ref):
    inv = 1.0 / jnp.maximum(cnt_ref[0][:, 0:1], 1.0)
    h = jnp.maximum(agg_ref[0] * inv + z0_ref[0], 0.0)
    b = jnp.where(pl.program_id(0) == 0, b_ref[0:1, :], b_ref[1:2, :])
    y_ref[...] = jnp.dot(h, wr_ref[0], preferred_element_type=_f32)
    z_ref[0] = jnp.dot(h, wt_ref[0], preferred_element_type=_f32) + b


_tc_comb_mm = pl.pallas_call(
    _comb_mm_body,
    grid=(NMP, N // BM),
    in_specs=[
        pl.BlockSpec((1, BM, D), lambda c, m: (c, m, 0)),
        pl.BlockSpec((1, BM, D), lambda c, m: (c, m, 0)),
        pl.BlockSpec((1, BM, D), lambda c, m: (c, m, 0)),
        pl.BlockSpec((1, D, D), lambda c, m: (c, 0, 0)),
        pl.BlockSpec((1, D, D), lambda c, m: (c, 0, 0)),
        pl.BlockSpec((NMP, D), lambda c, m: (0, 0)),
    ],
    out_specs=[
        pl.BlockSpec((BM, D), lambda c, m: (c * (N // BM) + m, 0)),
        pl.BlockSpec((1, BM, D), lambda c, m: (c, m, 0)),
    ],
    out_shape=[
        jax.ShapeDtypeStruct((NMP * N, D), _f32),
        jax.ShapeDtypeStruct((NMP, N, D), _f32),
    ],
)


def _fuse_body(agg_ref, cnt_ref, z1_ref, meta_ref, wqt_ref, bq_ref, o_ref):
    q = jnp.dot(meta_ref[...], wqt_ref[...],
                preferred_element_type=_f32) + bq_ref[...]
    inv0 = 1.0 / jnp.maximum(cnt_ref[0][:, 0:1], 1.0)
    inv1 = 1.0 / jnp.maximum(cnt_ref[1][:, 0:1], 1.0)
    h0 = jnp.maximum(agg_ref[0] * inv0 + z1_ref[0], 0.0)
    h1 = jnp.maximum(agg_ref[1] * inv1 + z1_ref[1], 0.0)
    scale = 1.0 / math.sqrt(D)
    s0 = jnp.sum(h0 * q[0:1, :], axis=1, keepdims=True) * scale
    s1 = jnp.sum(h1 * q[1:2, :], axis=1, keepdims=True) * scale
    w0 = 1.0 / (1.0 + jnp.exp(s1 - s0))
    o_ref[...] = w0 * h0 + (1.0 - w0) * h1


_tc_fuse = pl.pallas_call(
    _fuse_body,
    grid=(N // BM,),
    in_specs=[
        pl.BlockSpec((NMP, BM, D), lambda m: (0, m, 0)),
        pl.BlockSpec((NMP, BM, D), lambda m: (0, m, 0)),
        pl.BlockSpec((NMP, BM, D), lambda m: (0, m, 0)),
        pl.BlockSpec((NMP, 64), lambda m: (0, 0)),
        pl.BlockSpec((64, D), lambda m: (0, 0)),
        pl.BlockSpec((1, D), lambda m: (0, 0)),
    ],
    out_specs=pl.BlockSpec((BM, D), lambda m: (m, 0)),
    out_shape=jax.ShapeDtypeStruct((N, D), _f32),
)


def _prep_edges(ei, c):
    src = ei[0].astype(jnp.int32) + jnp.int32(c * N)
    dst = ei[1].astype(jnp.int32)
    pad = EPC - EDGES
    src = jnp.concatenate([src, jnp.zeros((pad,), jnp.int32)])
    dst = jnp.concatenate([dst, jnp.full((pad,), DUMP, jnp.int32)])
    return src.reshape(NTILES, CHUNKS, K), dst.reshape(NTILES, CHUNKS, K)


def kernel(E, edge_index0, eids0, edge_index1, eids1, metapath_emb,
           ifdropout, W_rel_0, W_root_0, b_0, W_rel_1, W_root_1, b_1,
           W_rel_2, W_root_2, b_2, W_rel_3, W_root_3, b_3, Wq, bq):
    eids_all = jnp.stack([eids0, eids1]).astype(jnp.int32)
    eids = jnp.stack([eids_all[:, s * WSTRIPE:s * WSTRIPE + GPT]
                      for s in range(NTILES)], axis=1)
    eids = eids.reshape(NMP, NTILES, GCH, K)

    s0, d0 = _prep_edges(edge_index0, 0)
    s1, d1 = _prep_edges(edge_index1, 1)
    src_r = jnp.stack([s0, s1])
    dst_r = jnp.stack([d0, d1])

    Wr0 = jnp.stack([W_rel_0[0], W_rel_2[0]])
    Wt0 = jnp.stack([W_root_0, W_root_2])
    bb0 = jnp.stack([b_0, b_2])
    Wr1 = jnp.stack([W_rel_1[0], W_rel_3[0]])
    Wt1 = jnp.stack([W_root_1, W_root_3])
    bb1 = jnp.stack([b_1, b_3])
    WqT = Wq.T
    bq2 = bq.reshape(1, D)

    h0, cnt = _sc_gather_count(E, eids, dst_r)
    y0, z0 = _tc_mm(h0, Wr0, Wt0, bb0)
    agg0 = _sc_edge_agg(y0, src_r, dst_r)
    y1, z1 = _tc_comb_mm(agg0, cnt, z0, Wr1, Wt1, bb1)
    agg1 = _sc_edge_agg(y1, src_r, dst_r)
    return _tc_fuse(agg1, cnt, z1, metapath_emb, WqT, bq2)


# speedup vs baseline: 1.4679x; 1.0296x over previous
import functools
import math

import jax
import jax.numpy as jnp
from jax import lax
from jax.experimental import pallas as pl
from jax.experimental.pallas import tpu as pltpu
from jax.experimental.pallas import tpu_sc as plsc

N = 10000
EDGES = 320000
D = 128
NMP = 2
NSC = 2
NTILES = 16
K = 128
CHUNKS = 160
HALF = CHUNKS // 2
QC = 40
EPT = CHUNKS * K
EPC = NTILES * EPT
DUMP = N
ZSTRIPE = 632
NROWS = NTILES * ZSTRIPE
WSTRIPE = 624
GCH = 5
GPT = GCH * K
BM = 2000

_f32 = jnp.float32
_MESH = dict(core_axis_name="c", subcore_axis_name="s",
             num_cores=NSC, num_subcores=NTILES)


def _gather_count_body(e_hbm, eids_hbm, dst_hbm, h0_hbm, cnt_hbm,
                       cnt_sh, idx_v, rows_v, dst_v, ones_v, sem):
    cid = lax.axis_index("c")
    sid = lax.axis_index("s")

    @pl.loop(0, K * (D // 16))
    def _fill(i):
        r = i // (D // 16)
        col = pl.ds((i % (D // 16)) * 16, 16)
        rows_v[r, col] = jnp.zeros((16,), _f32)
        ones_v[r, col] = jnp.ones((16,), _f32)

    zbase = sid * ZSTRIPE

    @pl.loop(0, ZSTRIPE // K)
    def _zstripe(k):
        pltpu.sync_copy(rows_v, cnt_sh.at[pl.ds(zbase + k * K, K)])

    rem = ZSTRIPE - (ZSTRIPE // K) * K
    pltpu.sync_copy(rows_v.at[pl.ds(0, rem)],
                    cnt_sh.at[pl.ds(zbase + (ZSTRIPE // K) * K, rem)])

    pltpu.sync_copy(eids_hbm.at[cid, sid], idx_v)
    base = sid * WSTRIPE
    for j in range(GCH - 1):
        pltpu.async_copy(e_hbm.at[idx_v.at[j]], rows_v, sem).wait()
        pltpu.sync_copy(rows_v, h0_hbm.at[cid, pl.ds(base + j * K, K)])
    pltpu.async_copy(e_hbm.at[idx_v.at[GCH - 1]], rows_v, sem).wait()
    tail = WSTRIPE - (GCH - 1) * K

    @pl.when(sid < NTILES - 1)
    def _w_tail():
        pltpu.sync_copy(rows_v.at[pl.ds(0, tail)],
                        h0_hbm.at[cid, pl.ds(base + (GCH - 1) * K, tail)])

    @pl.when(sid == NTILES - 1)
    def _w_tail_last():
        pltpu.sync_copy(rows_v,
                        h0_hbm.at[cid, pl.ds(base + (GCH - 1) * K, K)])

    plsc.subcore_barrier()

    for h in range(2):
        pltpu.sync_copy(dst_hbm.at[cid, sid, pl.ds(h * HALF, HALF)], dst_v)

        @pl.loop(0, HALF // 8)
        def _count(g):
            for t in range(8):
                pltpu.async_copy(ones_v, cnt_sh.at[dst_v.at[8 * g + t]],
                                 sem, add=True)
            for t in range(8):
                pltpu.make_async_copy(ones_v, cnt_sh.at[dst_v.at[8 * g]],
                                      sem).wait()

    plsc.subcore_barrier()
    pltpu.sync_copy(cnt_sh.at[pl.ds(base, WSTRIPE)],
                    cnt_hbm.at[cid, pl.ds(base, WSTRIPE)])

    @pl.when(sid == NTILES - 1)
    def _w_cnt_last():
        pltpu.sync_copy(cnt_sh.at[pl.ds(NTILES * WSTRIPE, N - NTILES * WSTRIPE)],
                        cnt_hbm.at[cid, pl.ds(NTILES * WSTRIPE,
                                              N - NTILES * WSTRIPE)])


_sc_gather_count = functools.partial(
    pl.kernel,
    out_type=(jax.ShapeDtypeStruct((NMP, N, D), _f32),
              jax.ShapeDtypeStruct((NMP, N, D), _f32)),
    mesh=plsc.VectorSubcoreMesh(**_MESH),
    scratch_types=[
        pltpu.VMEM_SHARED((NROWS, D), _f32),
        pltpu.VMEM((GCH, K), jnp.int32),
        pltpu.VMEM((K, D), _f32),
        pltpu.VMEM((HALF, K), jnp.int32),
        pltpu.VMEM((K, D), _f32),
        pltpu.SemaphoreType.DMA,
    ],
)(_gather_count_body)


def _edge_agg_body(y_hbm, src_hbm, dst_hbm, agg_hbm,
                   agg_sh, src_v, dst_v, ba, bb, fbuf, sema, semb):
    cid = lax.axis_index("c")
    sid = lax.axis_index("s")

    @pl.loop(0, K * (D // 16))
    def _zfill(i):
        fbuf[i // (D // 16), pl.ds((i % (D // 16)) * 16, 16)] = (
            jnp.zeros((16,), _f32))

    zbase = sid * ZSTRIPE

    @pl.loop(0, ZSTRIPE // K)
    def _zstripe(k):
        pltpu.sync_copy(fbuf, agg_sh.at[pl.ds(zbase + k * K, K)])

    rem = ZSTRIPE - (ZSTRIPE // K) * K
    pltpu.sync_copy(fbuf.at[pl.ds(0, rem)],
                    agg_sh.at[pl.ds(zbase + (ZSTRIPE // K) * K, rem)])

    plsc.subcore_barrier()

    def _wait(buf, sem):
        pltpu.make_async_copy(y_hbm.at[pl.ds(0, K)], buf, sem).wait()

    def _expand(buf):
        @pl.loop(0, K * (D // 32))
        def _cv(i):
            r = i // (D // 32)
            c = i % (D // 32)
            xi = buf[r, pl.ds(c * 16, 16)]
            sh = jnp.full((16,), 16, jnp.int32)
            msk = jnp.full((16,), -65536, jnp.int32)
            lo = lax.bitcast_convert_type(jnp.left_shift(xi, sh), _f32)
            hi = lax.bitcast_convert_type(jnp.bitwise_and(xi, msk), _f32)
            fbuf[r, pl.ds(c * 32, 16)] = lo
            fbuf[r, pl.ds(c * 32 + 16, 16)] = hi

    def _scat(j):
        pltpu.sync_copy(fbuf, agg_sh.at[dst_v.at[j]], add=True)

    for q in range(CHUNKS // QC):
        pltpu.sync_copy(src_hbm.at[cid, sid, pl.ds(q * QC, QC)], src_v)
        pltpu.sync_copy(dst_hbm.at[cid, sid, pl.ds(q * QC, QC)], dst_v)
        pltpu.async_copy(y_hbm.at[src_v.at[0]], ba, sema)

        @pl.loop(0, QC // 2)
        def _pairs(p):
            j0 = 2 * p
            pltpu.async_copy(y_hbm.at[src_v.at[j0 + 1]], bb, semb)
            _wait(ba, sema)
            _expand(ba)
            _scat(j0)
            jn = jnp.minimum(j0 + 2, QC - 1)
            pltpu.async_copy(y_hbm.at[src_v.at[jn]], ba, sema)
            _wait(bb, semb)
            _expand(bb)
            _scat(j0 + 1)

        _wait(ba, sema)

    plsc.subcore_barrier()
    pltpu.sync_copy(agg_sh.at[pl.ds(sid * WSTRIPE, WSTRIPE)],
                    agg_hbm.at[cid, pl.ds(sid * WSTRIPE, WSTRIPE)])

    @pl.when(sid == NTILES - 1)
    def _w_last():
        pltpu.sync_copy(agg_sh.at[pl.ds(NTILES * WSTRIPE, N - NTILES * WSTRIPE)],
                        agg_hbm.at[cid, pl.ds(NTILES * WSTRIPE,
                                              N - NTILES * WSTRIPE)])


_sc_edge_agg = functools.partial(
    pl.kernel,
    out_type=jax.ShapeDtypeStruct((NMP, N, D), _f32),
    mesh=plsc.VectorSubcoreMesh(**_MESH),
    scratch_types=[
        pltpu.VMEM_SHARED((NROWS, D), _f32),
        pltpu.VMEM((QC, K), jnp.int32),
        pltpu.VMEM((QC, K), jnp.int32),
        pltpu.VMEM((K, D // 2), jnp.int32),
        pltpu.VMEM((K, D // 2), jnp.int32),
        pltpu.VMEM((K, D), _f32),
        pltpu.SemaphoreType.DMA,
        pltpu.SemaphoreType.DMA,
    ],
    compiler_params=pltpu.CompilerParams(use_tc_tiling_on_sc=False),
)(_edge_agg_body)


def _mm_body(h_ref, wr_ref, wt_ref, b_ref, y_ref, z_ref):
    h = h_ref[0]
    b = jnp.where(pl.program_id(0) == 0, b_ref[0:1, :], b_ref[1:2, :])
    y_ref[...] = jnp.dot(h, wr_ref[0],
                         preferred_element_type=_f32).astype(jnp.bfloat16)
    z_ref[0] = jnp.dot(h, wt_ref[0], preferred_element_type=_f32) + b


_tc_mm = pl.pallas_call(
    _mm_body,
    grid=(NMP, N // BM),
    in_specs=[
        pl.BlockSpec((1, BM, D), lambda c, m: (c, m, 0)),
        pl.BlockSpec((1, D, D), lambda c, m: (c, 0, 0)),
        pl.BlockSpec((1, D, D), lambda c, m: (c, 0, 0)),
        pl.BlockSpec((NMP, D), lambda c, m: (0, 0)),
    ],
    out_specs=[
        pl.BlockSpec((BM, D), lambda c, m: (c * (N // BM) + m, 0)),
        pl.BlockSpec((1, BM, D), lambda c, m: (c, m, 0)),
    ],
    out_shape=[
        jax.ShapeDtypeStruct((NMP * N, D), jnp.bfloat16),
        jax.ShapeDtypeStruct((NMP, N, D), _f32),
    ],
)


def _comb_mm_body(agg_ref, cnt_ref, z0_ref, wr_ref, wt_ref, b_ref,
                  y_ref, z_ref):
    inv = 1.0 / jnp.maximum(cnt_ref[0][:, 0:1], 1.0)
    h = jnp.maximum(agg_ref[0] * inv + z0_ref[0], 0.0)
    b = jnp.where(pl.program_id(0) == 0, b_ref[0:1, :], b_ref[1:2, :])
    y_ref[...] = jnp.dot(h, wr_ref[0],
                         preferred_element_type=_f32).astype(jnp.bfloat16)
    z_ref[0] = jnp.dot(h, wt_ref[0], preferred_element_type=_f32) + b


_tc_comb_mm = pl.pallas_call(
    _comb_mm_body,
    grid=(NMP, N // BM),
    in_specs=[
        pl.BlockSpec((1, BM, D), lambda c, m: (c, m, 0)),
        pl.BlockSpec((1, BM, D), lambda c, m: (c, m, 0)),
        pl.BlockSpec((1, BM, D), lambda c, m: (c, m, 0)),
        pl.BlockSpec((1, D, D), lambda c, m: (c, 0, 0)),
        pl.BlockSpec((1, D, D), lambda c, m: (c, 0, 0)),
        pl.BlockSpec((NMP, D), lambda c, m: (0, 0)),
    ],
    out_specs=[
        pl.BlockSpec((BM, D), lambda c, m: (c * (N // BM) + m, 0)),
        pl.BlockSpec((1, BM, D), lambda c, m: (c, m, 0)),
    ],
    out_shape=[
        jax.ShapeDtypeStruct((NMP * N, D), jnp.bfloat16),
        jax.ShapeDtypeStruct((NMP, N, D), _f32),
    ],
)


def _fuse_body(agg_ref, cnt_ref, z1_ref, meta_ref, wqt_ref, bq_ref, o_ref):
    q = jnp.dot(meta_ref[...], wqt_ref[...],
                preferred_element_type=_f32) + bq_ref[...]
    inv0 = 1.0 / jnp.maximum(cnt_ref[0][:, 0:1], 1.0)
    inv1 = 1.0 / jnp.maximum(cnt_ref[1][:, 0:1], 1.0)
    h0 = jnp.maximum(agg_ref[0] * inv0 + z1_ref[0], 0.0)
    h1 = jnp.maximum(agg_ref[1] * inv1 + z1_ref[1], 0.0)
    scale = 1.0 / math.sqrt(D)
    s0 = jnp.sum(h0 * q[0:1, :], axis=1, keepdims=True) * scale
    s1 = jnp.sum(h1 * q[1:2, :], axis=1, keepdims=True) * scale
    w0 = 1.0 / (1.0 + jnp.exp(s1 - s0))
    o_ref[...] = w0 * h0 + (1.0 - w0) * h1


_tc_fuse = pl.pallas_call(
    _fuse_body,
    grid=(N // BM,),
    in_specs=[
        pl.BlockSpec((NMP, BM, D), lambda m: (0, m, 0)),
        pl.BlockSpec((NMP, BM, D), lambda m: (0, m, 0)),
        pl.BlockSpec((NMP, BM, D), lambda m: (0, m, 0)),
        pl.BlockSpec((NMP, 64), lambda m: (0, 0)),
        pl.BlockSpec((64, D), lambda m: (0, 0)),
        pl.BlockSpec((1, D), lambda m: (0, 0)),
    ],
    out_specs=pl.BlockSpec((BM, D), lambda m: (m, 0)),
    out_shape=jax.ShapeDtypeStruct((N, D), _f32),
)


def _prep_edges(ei, c):
    src = ei[0].astype(jnp.int32) + jnp.int32(c * N)
    dst = ei[1].astype(jnp.int32)
    pad = EPC - EDGES
    src = jnp.concatenate([src, jnp.zeros((pad,), jnp.int32)])
    dst = jnp.concatenate([dst, jnp.full((pad,), DUMP, jnp.int32)])
    return src.reshape(NTILES, CHUNKS, K), dst.reshape(NTILES, CHUNKS, K)


def kernel(E, edge_index0, eids0, edge_index1, eids1, metapath_emb,
           ifdropout, W_rel_0, W_root_0, b_0, W_rel_1, W_root_1, b_1,
           W_rel_2, W_root_2, b_2, W_rel_3, W_root_3, b_3, Wq, bq):
    eids_all = jnp.stack([eids0, eids1]).astype(jnp.int32)
    eids = jnp.stack([eids_all[:, s * WSTRIPE:s * WSTRIPE + GPT]
                      for s in range(NTILES)], axis=1)
    eids = eids.reshape(NMP, NTILES, GCH, K)

    s0, d0 = _prep_edges(edge_index0, 0)
    s1, d1 = _prep_edges(edge_index1, 1)
    src_r = jnp.stack([s0, s1])
    dst_r = jnp.stack([d0, d1])

    qv = jnp.arange(D)
    perm = 32 * (qv // 32) + (qv % 32) // 2 + 16 * ((qv % 32) % 2)
    Wr0 = jnp.stack([W_rel_0[0][:, perm], W_rel_2[0][:, perm]])
    Wt0 = jnp.stack([W_root_0, W_root_2])
    bb0 = jnp.stack([b_0, b_2])
    Wr1 = jnp.stack([W_rel_1[0][:, perm], W_rel_3[0][:, perm]])
    Wt1 = jnp.stack([W_root_1, W_root_3])
    bb1 = jnp.stack([b_1, b_3])
    WqT = Wq.T
    bq2 = bq.reshape(1, D)

    def _pack(y):
        return lax.bitcast_convert_type(
            y.reshape(NMP * N, D // 2, 2), jnp.int32)

    h0, cnt = _sc_gather_count(E, eids, dst_r)
    y0, z0 = _tc_mm(h0, Wr0, Wt0, bb0)
    agg0 = _sc_edge_agg(_pack(y0), src_r, dst_r)
    y1, z1 = _tc_comb_mm(agg0, cnt, z0, Wr1, Wt1, bb1)
    agg1 = _sc_edge_agg(_pack(y1), src_r, dst_r)
    return _tc_fuse(agg1, cnt, z1, metapath_emb, WqT, bq2)


# speedup vs baseline: 1.4683x; 1.0003x over previous
import functools
import math

import jax
import jax.numpy as jnp
from jax import lax
from jax.experimental import pallas as pl
from jax.experimental.pallas import tpu as pltpu
from jax.experimental.pallas import tpu_sc as plsc

N = 10000
EDGES = 320000
D = 128
NMP = 2
NSC = 2
NTILES = 16
K = 128
CHUNKS = 160
HALF = CHUNKS // 2
QC = 40
EPT = CHUNKS * K
EPC = NTILES * EPT
DUMP = N
ZSTRIPE = 632
NROWS = NTILES * ZSTRIPE
WSTRIPE = 624
GCH = 5
GPT = GCH * K
BM = 2000

_f32 = jnp.float32
_MESH = dict(core_axis_name="c", subcore_axis_name="s",
             num_cores=NSC, num_subcores=NTILES)


def _gather_count_body(e_hbm, eids_hbm, dst_hbm, h0_hbm, cnt_hbm,
                       cnt_sh, idx_v, rows_v, dst_v, ones_v, sem):
    cid = lax.axis_index("c")
    sid = lax.axis_index("s")

    @pl.loop(0, K * (D // 16))
    def _fill(i):
        r = i // (D // 16)
        col = pl.ds((i % (D // 16)) * 16, 16)
        rows_v[r, col] = jnp.zeros((16,), _f32)
        ones_v[r, col] = jnp.ones((16,), _f32)

    zbase = sid * ZSTRIPE

    @pl.loop(0, ZSTRIPE // K)
    def _zstripe(k):
        pltpu.sync_copy(rows_v, cnt_sh.at[pl.ds(zbase + k * K, K)])

    rem = ZSTRIPE - (ZSTRIPE // K) * K
    pltpu.sync_copy(rows_v.at[pl.ds(0, rem)],
                    cnt_sh.at[pl.ds(zbase + (ZSTRIPE // K) * K, rem)])

    pltpu.sync_copy(eids_hbm.at[cid, sid], idx_v)
    base = sid * WSTRIPE
    for j in range(GCH - 1):
        pltpu.async_copy(e_hbm.at[idx_v.at[j]], rows_v, sem).wait()
        pltpu.sync_copy(rows_v, h0_hbm.at[cid, pl.ds(base + j * K, K)])
    pltpu.async_copy(e_hbm.at[idx_v.at[GCH - 1]], rows_v, sem).wait()
    tail = WSTRIPE - (GCH - 1) * K

    @pl.when(sid < NTILES - 1)
    def _w_tail():
        pltpu.sync_copy(rows_v.at[pl.ds(0, tail)],
                        h0_hbm.at[cid, pl.ds(base + (GCH - 1) * K, tail)])

    @pl.when(sid == NTILES - 1)
    def _w_tail_last():
        pltpu.sync_copy(rows_v,
                        h0_hbm.at[cid, pl.ds(base + (GCH - 1) * K, K)])

    plsc.subcore_barrier()

    for h in range(2):
        pltpu.sync_copy(dst_hbm.at[cid, sid, pl.ds(h * HALF, HALF)], dst_v)

        @pl.loop(0, HALF // 8)
        def _count(g):
            for t in range(8):
                pltpu.async_copy(ones_v, cnt_sh.at[dst_v.at[8 * g + t]],
                                 sem, add=True)
            for t in range(8):
                pltpu.make_async_copy(ones_v, cnt_sh.at[dst_v.at[8 * g]],
                                      sem).wait()

    plsc.subcore_barrier()
    pltpu.sync_copy(cnt_sh.at[pl.ds(base, WSTRIPE)],
                    cnt_hbm.at[cid, pl.ds(base, WSTRIPE)])

    @pl.when(sid == NTILES - 1)
    def _w_cnt_last():
        pltpu.sync_copy(cnt_sh.at[pl.ds(NTILES * WSTRIPE, N - NTILES * WSTRIPE)],
                        cnt_hbm.at[cid, pl.ds(NTILES * WSTRIPE,
                                              N - NTILES * WSTRIPE)])


_sc_gather_count = functools.partial(
    pl.kernel,
    out_type=(jax.ShapeDtypeStruct((NMP, N, D), _f32),
              jax.ShapeDtypeStruct((NMP, N, D), _f32)),
    mesh=plsc.VectorSubcoreMesh(**_MESH),
    scratch_types=[
        pltpu.VMEM_SHARED((NROWS, D), _f32),
        pltpu.VMEM((GCH, K), jnp.int32),
        pltpu.VMEM((K, D), _f32),
        pltpu.VMEM((HALF, K), jnp.int32),
        pltpu.VMEM((K, D), _f32),
        pltpu.SemaphoreType.DMA,
    ],
)(_gather_count_body)


def _edge_agg_body(y_hbm, src_hbm, dst_hbm, agg_hbm,
                   agg_sh, src_v, dst_v, ba, bb, fbuf, sema, semb):
    cid = lax.axis_index("c")
    sid = lax.axis_index("s")

    @pl.loop(0, K * (D // 16))
    def _zfill(i):
        fbuf[i // (D // 16), pl.ds((i % (D // 16)) * 16, 16)] = (
            jnp.zeros((16,), _f32))

    zbase = sid * ZSTRIPE

    @pl.loop(0, ZSTRIPE // K)
    def _zstripe(k):
        pltpu.sync_copy(fbuf, agg_sh.at[pl.ds(zbase + k * K, K)])

    rem = ZSTRIPE - (ZSTRIPE // K) * K
    pltpu.sync_copy(fbuf.at[pl.ds(0, rem)],
                    agg_sh.at[pl.ds(zbase + (ZSTRIPE // K) * K, rem)])

    plsc.subcore_barrier()

    def _wait(buf, sem):
        pltpu.make_async_copy(y_hbm.at[pl.ds(0, K)], buf, sem).wait()

    def _expand(buf):
        @pl.loop(0, K * (D // 32), unroll=4)
        def _cv(i):
            r = i // (D // 32)
            c = i % (D // 32)
            xi = buf[r, pl.ds(c * 16, 16)]
            sh = jnp.full((16,), 16, jnp.int32)
            msk = jnp.full((16,), -65536, jnp.int32)
            lo = lax.bitcast_convert_type(jnp.left_shift(xi, sh), _f32)
            hi = lax.bitcast_convert_type(jnp.bitwise_and(xi, msk), _f32)
            fbuf[r, pl.ds(c * 32, 16)] = lo
            fbuf[r, pl.ds(c * 32 + 16, 16)] = hi

    def _scat(j):
        pltpu.sync_copy(fbuf, agg_sh.at[dst_v.at[j]], add=True)

    for q in range(CHUNKS // QC):
        pltpu.sync_copy(src_hbm.at[cid, sid, pl.ds(q * QC, QC)], src_v)
        pltpu.sync_copy(dst_hbm.at[cid, sid, pl.ds(q * QC, QC)], dst_v)
        pltpu.async_copy(y_hbm.at[src_v.at[0]], ba, sema)

        @pl.loop(0, QC // 2)
        def _pairs(p):
            j0 = 2 * p
            pltpu.async_copy(y_hbm.at[src_v.at[j0 + 1]], bb, semb)
            _wait(ba, sema)
            _expand(ba)
            _scat(j0)
            jn = jnp.minimum(j0 + 2, QC - 1)
            pltpu.async_copy(y_hbm.at[src_v.at[jn]], ba, sema)
            _wait(bb, semb)
            _expand(bb)
            _scat(j0 + 1)

        _wait(ba, sema)

    plsc.subcore_barrier()
    pltpu.sync_copy(agg_sh.at[pl.ds(sid * WSTRIPE, WSTRIPE)],
                    agg_hbm.at[cid, pl.ds(sid * WSTRIPE, WSTRIPE)])

    @pl.when(sid == NTILES - 1)
    def _w_last():
        pltpu.sync_copy(agg_sh.at[pl.ds(NTILES * WSTRIPE, N - NTILES * WSTRIPE)],
                        agg_hbm.at[cid, pl.ds(NTILES * WSTRIPE,
                                              N - NTILES * WSTRIPE)])


_sc_edge_agg = functools.partial(
    pl.kernel,
    out_type=jax.ShapeDtypeStruct((NMP, N, D), _f32),
    mesh=plsc.VectorSubcoreMesh(**_MESH),
    scratch_types=[
        pltpu.VMEM_SHARED((NROWS, D), _f32),
        pltpu.VMEM((QC, K), jnp.int32),
        pltpu.VMEM((QC, K), jnp.int32),
        pltpu.VMEM((K, D // 2), jnp.int32),
        pltpu.VMEM((K, D // 2), jnp.int32),
        pltpu.VMEM((K, D), _f32),
        pltpu.SemaphoreType.DMA,
        pltpu.SemaphoreType.DMA,
    ],
    compiler_params=pltpu.CompilerParams(use_tc_tiling_on_sc=False),
)(_edge_agg_body)


def _mm_body(h_ref, wr_ref, wt_ref, b_ref, y_ref, z_ref):
    h = h_ref[0]
    b = jnp.where(pl.program_id(0) == 0, b_ref[0:1, :], b_ref[1:2, :])
    y_ref[...] = jnp.dot(h, wr_ref[0],
                         preferred_element_type=_f32).astype(jnp.bfloat16)
    z_ref[0] = jnp.dot(h, wt_ref[0], preferred_element_type=_f32) + b


_tc_mm = pl.pallas_call(
    _mm_body,
    grid=(NMP, N // BM),
    in_specs=[
        pl.BlockSpec((1, BM, D), lambda c, m: (c, m, 0)),
        pl.BlockSpec((1, D, D), lambda c, m: (c, 0, 0)),
        pl.BlockSpec((1, D, D), lambda c, m: (c, 0, 0)),
        pl.BlockSpec((NMP, D), lambda c, m: (0, 0)),
    ],
    out_specs=[
        pl.BlockSpec((BM, D), lambda c, m: (c * (N // BM) + m, 0)),
        pl.BlockSpec((1, BM, D), lambda c, m: (c, m, 0)),
    ],
    out_shape=[
        jax.ShapeDtypeStruct((NMP * N, D), jnp.bfloat16),
        jax.ShapeDtypeStruct((NMP, N, D), _f32),
    ],
)


def _comb_mm_body(agg_ref, cnt_ref, z0_ref, wr_ref, wt_ref, b_ref,
                  y_ref, z_ref):
    inv = 1.0 / jnp.maximum(cnt_ref[0][:, 0:1], 1.0)
    h = jnp.maximum(agg_ref[0] * inv + z0_ref[0], 0.0)
    b = jnp.where(pl.program_id(0) == 0, b_ref[0:1, :], b_ref[1:2, :])
    y_ref[...] = jnp.dot(h, wr_ref[0],
                         preferred_element_type=_f32).astype(jnp.bfloat16)
    z_ref[0] = jnp.dot(h, wt_ref[0], preferred_element_type=_f32) + b


_tc_comb_mm = pl.pallas_call(
    _comb_mm_body,
    grid=(NMP, N // BM),
    in_specs=[
        pl.BlockSpec((1, BM, D), lambda c, m: (c, m, 0)),
        pl.BlockSpec((1, BM, D), lambda c, m: (c, m, 0)),
        pl.BlockSpec((1, BM, D), lambda c, m: (c, m, 0)),
        pl.BlockSpec((1, D, D), lambda c, m: (c, 0, 0)),
        pl.BlockSpec((1, D, D), lambda c, m: (c, 0, 0)),
        pl.BlockSpec((NMP, D), lambda c, m: (0, 0)),
    ],
    out_specs=[
        pl.BlockSpec((BM, D), lambda c, m: (c * (N // BM) + m, 0)),
        pl.BlockSpec((1, BM, D), lambda c, m: (c, m, 0)),
    ],
    out_shape=[
        jax.ShapeDtypeStruct((NMP * N, D), jnp.bfloat16),
        jax.ShapeDtypeStruct((NMP, N, D), _f32),
    ],
)


def _fuse_body(agg_ref, cnt_ref, z1_ref, meta_ref, wqt_ref, bq_ref, o_ref):
    q = jnp.dot(meta_ref[...], wqt_ref[...],
                preferred_element_type=_f32) + bq_ref[...]
    inv0 = 1.0 / jnp.maximum(cnt_ref[0][:, 0:1], 1.0)
    inv1 = 1.0 / jnp.maximum(cnt_ref[1][:, 0:1], 1.0)
    h0 = jnp.maximum(agg_ref[0] * inv0 + z1_ref[0], 0.0)
    h1 = jnp.maximum(agg_ref[1] * inv1 + z1_ref[1], 0.0)
    scale = 1.0 / math.sqrt(D)
    s0 = jnp.sum(h0 * q[0:1, :], axis=1, keepdims=True) * scale
    s1 = jnp.sum(h1 * q[1:2, :], axis=1, keepdims=True) * scale
    w0 = 1.0 / (1.0 + jnp.exp(s1 - s0))
    o_ref[...] = w0 * h0 + (1.0 - w0) * h1


_tc_fuse = pl.pallas_call(
    _fuse_body,
    grid=(N // BM,),
    in_specs=[
        pl.BlockSpec((NMP, BM, D), lambda m: (0, m, 0)),
        pl.BlockSpec((NMP, BM, D), lambda m: (0, m, 0)),
        pl.BlockSpec((NMP, BM, D), lambda m: (0, m, 0)),
        pl.BlockSpec((NMP, 64), lambda m: (0, 0)),
        pl.BlockSpec((64, D), lambda m: (0, 0)),
        pl.BlockSpec((1, D), lambda m: (0, 0)),
    ],
    out_specs=pl.BlockSpec((BM, D), lambda m: (m, 0)),
    out_shape=jax.ShapeDtypeStruct((N, D), _f32),
)


def _prep_edges(ei, c):
    src = ei[0].astype(jnp.int32) + jnp.int32(c * N)
    dst = ei[1].astype(jnp.int32)
    pad = EPC - EDGES
    src = jnp.concatenate([src, jnp.zeros((pad,), jnp.int32)])
    dst = jnp.concatenate([dst, jnp.full((pad,), DUMP, jnp.int32)])
    return src.reshape(NTILES, CHUNKS, K), dst.reshape(NTILES, CHUNKS, K)


def kernel(E, edge_index0, eids0, edge_index1, eids1, metapath_emb,
           ifdropout, W_rel_0, W_root_0, b_0, W_rel_1, W_root_1, b_1,
           W_rel_2, W_root_2, b_2, W_rel_3, W_root_3, b_3, Wq, bq):
    eids_all = jnp.stack([eids0, eids1]).astype(jnp.int32)
    eids = jnp.stack([eids_all[:, s * WSTRIPE:s * WSTRIPE + GPT]
                      for s in range(NTILES)], axis=1)
    eids = eids.reshape(NMP, NTILES, GCH, K)

    s0, d0 = _prep_edges(edge_index0, 0)
    s1, d1 = _prep_edges(edge_index1, 1)
    src_r = jnp.stack([s0, s1])
    dst_r = jnp.stack([d0, d1])

    qv = jnp.arange(D)
    perm = 32 * (qv // 32) + (qv % 32) // 2 + 16 * ((qv % 32) % 2)
    Wr0 = jnp.stack([W_rel_0[0][:, perm], W_rel_2[0][:, perm]])
    Wt0 = jnp.stack([W_root_0, W_root_2])
    bb0 = jnp.stack([b_0, b_2])
    Wr1 = jnp.stack([W_rel_1[0][:, perm], W_rel_3[0][:, perm]])
    Wt1 = jnp.stack([W_root_1, W_root_3])
    bb1 = jnp.stack([b_1, b_3])
    WqT = Wq.T
    bq2 = bq.reshape(1, D)

    def _pack(y):
        return lax.bitcast_convert_type(
            y.reshape(NMP * N, D // 2, 2), jnp.int32)

    h0, cnt = _sc_gather_count(E, eids, dst_r)
    y0, z0 = _tc_mm(h0, Wr0, Wt0, bb0)
    agg0 = _sc_edge_agg(_pack(y0), src_r, dst_r)
    y1, z1 = _tc_comb_mm(agg0, cnt, z0, Wr1, Wt1, bb1)
    agg1 = _sc_edge_agg(_pack(y1), src_r, dst_r)
    return _tc_fuse(agg1, cnt, z1, metapath_emb, WqT, bq2)
